# baseline (speedup 1.0000x reference)
import functools

import jax
import jax.numpy as jnp
from jax import lax
from jax.experimental import pallas as pl
from jax.experimental.pallas import tpu as pltpu
from jax.experimental.pallas import tpu_sc as plsc

LANES = 16
EMBED_DIM = 32
NUM_CORES = 2
NUM_SUBCORES = 16
NUM_WORKERS = NUM_CORES * NUM_SUBCORES


def _mf_body(users_hbm, items_hbm, user_table_hbm, item_table_hbm, out_hbm,
             uidx_v, iidx_v, urows_v, irows_v, out_v, sem_u, sem_i):
    batch = users_hbm.shape[0]
    bpw = batch // NUM_WORKERS
    wid = lax.axis_index("s") * NUM_CORES + lax.axis_index("c")
    base = wid * bpw

    pltpu.sync_copy(users_hbm.at[pl.ds(base, bpw)], uidx_v)
    pltpu.sync_copy(items_hbm.at[pl.ds(base, bpw)], iidx_v)
    cu = pltpu.async_copy(user_table_hbm.at[uidx_v], urows_v, sem_u)
    ci = pltpu.async_copy(item_table_hbm.at[iidx_v], irows_v, sem_i)
    cu.wait()
    ci.wait()

    lanes = lax.iota(jnp.int32, LANES)

    def group(g, carry):
        rows = g * LANES + lanes
        acc = jnp.zeros((LANES,), jnp.float32)
        for d in range(EMBED_DIM):
            dcol = jnp.full((LANES,), d, jnp.int32)
            uc = plsc.load_gather(urows_v, [rows, dcol])
            ic = plsc.load_gather(irows_v, [rows, dcol])
            acc = acc + uc * ic
        out_v[pl.ds(g * LANES, LANES)] = acc
        return carry

    lax.fori_loop(0, bpw // LANES, group, 0)
    pltpu.sync_copy(out_v, out_hbm.at[pl.ds(base, bpw)])


@jax.jit
def kernel(users, items, user_table, item_table):
    users = users.astype(jnp.int32)
    items = items.astype(jnp.int32)
    batch = users.shape[0]
    bpw = batch // NUM_WORKERS
    mesh = plsc.VectorSubcoreMesh(core_axis_name="c", subcore_axis_name="s")
    run = pl.kernel(
        _mf_body,
        out_type=jax.ShapeDtypeStruct((batch,), jnp.float32),
        mesh=mesh,
        scratch_types=[
            pltpu.VMEM((bpw,), jnp.int32),
            pltpu.VMEM((bpw,), jnp.int32),
            pltpu.VMEM((bpw, EMBED_DIM), jnp.float32),
            pltpu.VMEM((bpw, EMBED_DIM), jnp.float32),
            pltpu.VMEM((bpw,), jnp.float32),
            pltpu.SemaphoreType.DMA,
            pltpu.SemaphoreType.DMA,
        ],
        compiler_params=pltpu.CompilerParams(
            needs_layout_passes=False, use_tc_tiling_on_sc=False),
    )
    return run(users, items, user_table, item_table)


# speedup vs baseline: 3.8529x; 3.8529x over previous
import functools

import jax
import jax.numpy as jnp
from jax import lax
from jax.experimental import pallas as pl
from jax.experimental.pallas import tpu as pltpu
from jax.experimental.pallas import tpu_sc as plsc

LANES = 16
EMBED_DIM = 32
NUM_CORES = 2
NUM_SUBCORES = 16
NUM_WORKERS = NUM_CORES * NUM_SUBCORES
WAVE = 4
NBUF = 2
PANEL = 128


def _mf_body(users_hbm, items_hbm, ut_hbm, it_hbm, out_hbm,
             uidx_v, iidx_v, upan_v, ipan_v, ucols_v, icols_v, out_v,
             sem_u, sem_i):
    batch = users_hbm.shape[0]
    bpw = batch // NUM_WORKERS
    wid = lax.axis_index("s") * NUM_CORES + lax.axis_index("c")
    base = wid * bpw
    n_waves = bpw // WAVE

    pltpu.sync_copy(users_hbm.at[pl.ds(base, bpw)], uidx_v.at[pl.ds(0, bpw)])
    pltpu.sync_copy(items_hbm.at[pl.ds(base, bpw)], iidx_v.at[pl.ds(0, bpw)])

    d16 = lax.iota(jnp.int32, LANES)

    def fire(w, slot):
        uvec = uidx_v[pl.ds(w * WAVE, LANES)]
        ivec = iidx_v[pl.ds(w * WAVE, LANES)]
        for j in range(WAVE):
            uc0 = pl.multiple_of((uvec[j] >> 7) << 7, PANEL)
            ic0 = pl.multiple_of((ivec[j] >> 7) << 7, PANEL)
            pltpu.async_copy(
                ut_hbm.at[:, pl.ds(uc0, PANEL)], upan_v.at[slot, j], sem_u)
            pltpu.async_copy(
                it_hbm.at[:, pl.ds(ic0, PANEL)], ipan_v.at[slot, j], sem_i)

    def drain_extract(w, slot):
        uvec = uidx_v[pl.ds(w * WAVE, LANES)]
        ivec = iidx_v[pl.ds(w * WAVE, LANES)]
        for j in range(WAVE):
            pltpu.make_async_copy(
                ut_hbm.at[:, pl.ds(0, PANEL)], upan_v.at[slot, j], sem_u).wait()
            pltpu.make_async_copy(
                it_hbm.at[:, pl.ds(0, PANEL)], ipan_v.at[slot, j], sem_i).wait()
        cbase = (w % (LANES // WAVE)) * WAVE
        for j in range(WAVE):
            uoff = jnp.full((LANES,), uvec[j] & 127, jnp.int32)
            ioff = jnp.full((LANES,), ivec[j] & 127, jnp.int32)
            for h in range(EMBED_DIM // LANES):
                drows = h * LANES + d16
                uval = plsc.load_gather(upan_v, [
                    jnp.full((LANES,), slot, jnp.int32),
                    jnp.full((LANES,), j, jnp.int32), drows, uoff])
                ival = plsc.load_gather(ipan_v, [
                    jnp.full((LANES,), slot, jnp.int32),
                    jnp.full((LANES,), j, jnp.int32), drows, ioff])
                ucols_v[pl.ds((cbase + j) * EMBED_DIM + h * LANES, LANES)] = uval
                icols_v[pl.ds((cbase + j) * EMBED_DIM + h * LANES, LANES)] = ival

    def dot16(w):
        gbase = (w - (LANES // WAVE - 1)) * WAVE
        acc = jnp.zeros((LANES,), jnp.float32)
        flat0 = d16 * EMBED_DIM
        for d in range(EMBED_DIM):
            uc = plsc.load_gather(ucols_v, [flat0 + d])
            ic = plsc.load_gather(icols_v, [flat0 + d])
            acc = acc + uc * ic
        out_v[pl.ds(gbase, LANES)] = acc

    fire(0, 0)

    def pipe(step, carry):
        w0 = step * NBUF
        for p in range(NBUF):
            w = w0 + p

            @pl.when(w + 1 < n_waves)
            def _():
                fire(w + 1, (p + 1) % NBUF)

            drain_extract(w, p)

            @pl.when(w % (LANES // WAVE) == (LANES // WAVE - 1))
            def _():
                dot16(w)
        return carry

    lax.fori_loop(0, n_waves // NBUF, pipe, 0)
    pltpu.sync_copy(out_v, out_hbm.at[pl.ds(base, bpw)])


@jax.jit
def kernel(users, items, user_table, item_table):
    users = users.astype(jnp.int32)
    items = items.astype(jnp.int32)
    ut = user_table.T
    it = item_table.T
    batch = users.shape[0]
    bpw = batch // NUM_WORKERS
    mesh = plsc.VectorSubcoreMesh(core_axis_name="c", subcore_axis_name="s")
    run = pl.kernel(
        _mf_body,
        out_type=jax.ShapeDtypeStruct((batch,), jnp.float32),
        mesh=mesh,
        scratch_types=[
            pltpu.VMEM((bpw + LANES,), jnp.int32),
            pltpu.VMEM((bpw + LANES,), jnp.int32),
            pltpu.VMEM((NBUF, WAVE, EMBED_DIM, PANEL), jnp.float32),
            pltpu.VMEM((NBUF, WAVE, EMBED_DIM, PANEL), jnp.float32),
            pltpu.VMEM((LANES * EMBED_DIM,), jnp.float32),
            pltpu.VMEM((LANES * EMBED_DIM,), jnp.float32),
            pltpu.VMEM((bpw,), jnp.float32),
            pltpu.SemaphoreType.DMA,
            pltpu.SemaphoreType.DMA,
        ],
        compiler_params=pltpu.CompilerParams(
            needs_layout_passes=False, use_tc_tiling_on_sc=True),
    )
    return run(users, items, ut, it)


# speedup vs baseline: 4.1475x; 1.0765x over previous
import functools

import jax
import jax.numpy as jnp
from jax import lax
from jax.experimental import pallas as pl
from jax.experimental.pallas import tpu as pltpu
from jax.experimental.pallas import tpu_sc as plsc

LANES = 16
EMBED_DIM = 32
NUM_CORES = 2
NUM_SUBCORES = 16
NUM_WORKERS = NUM_CORES * NUM_SUBCORES
WAVE = 4
NBUF = 3
PANEL = 128


def _mf_body(users_hbm, items_hbm, ut_hbm, it_hbm, out_hbm,
             uidx_v, iidx_v, upan_v, ipan_v, ucols_v, icols_v, out_v,
             *sems):
    sem_u = sems[:NBUF]
    sem_i = sems[NBUF:]
    batch = users_hbm.shape[0]
    bpw = batch // NUM_WORKERS
    wid = lax.axis_index("s") * NUM_CORES + lax.axis_index("c")
    base = wid * bpw
    n_waves = bpw // WAVE

    pltpu.sync_copy(users_hbm.at[pl.ds(base, bpw)], uidx_v.at[pl.ds(0, bpw)])
    pltpu.sync_copy(items_hbm.at[pl.ds(base, bpw)], iidx_v.at[pl.ds(0, bpw)])

    d16 = lax.iota(jnp.int32, LANES)

    def fire(w, slot):
        uvec = uidx_v[pl.ds(w * WAVE, LANES)]
        ivec = iidx_v[pl.ds(w * WAVE, LANES)]
        for j in range(WAVE):
            uc0 = pl.multiple_of((uvec[j] >> 7) << 7, PANEL)
            ic0 = pl.multiple_of((ivec[j] >> 7) << 7, PANEL)
            pltpu.async_copy(
                ut_hbm.at[:, pl.ds(uc0, PANEL)], upan_v.at[slot, j], sem_u[slot])
            pltpu.async_copy(
                it_hbm.at[:, pl.ds(ic0, PANEL)], ipan_v.at[slot, j], sem_i[slot])

    def drain_extract(w, slot):
        uvec = uidx_v[pl.ds(w * WAVE, LANES)]
        ivec = iidx_v[pl.ds(w * WAVE, LANES)]
        for j in range(WAVE):
            pltpu.make_async_copy(
                ut_hbm.at[:, pl.ds(0, PANEL)], upan_v.at[slot, j],
                sem_u[slot]).wait()
            pltpu.make_async_copy(
                it_hbm.at[:, pl.ds(0, PANEL)], ipan_v.at[slot, j],
                sem_i[slot]).wait()
        cbase = (w % (LANES // WAVE)) * WAVE
        for j in range(WAVE):
            uoff = jnp.full((LANES,), uvec[j] & 127, jnp.int32)
            ioff = jnp.full((LANES,), ivec[j] & 127, jnp.int32)
            for h in range(EMBED_DIM // LANES):
                drows = h * LANES + d16
                uval = plsc.load_gather(upan_v, [
                    jnp.full((LANES,), slot, jnp.int32),
                    jnp.full((LANES,), j, jnp.int32), drows, uoff])
                ival = plsc.load_gather(ipan_v, [
                    jnp.full((LANES,), slot, jnp.int32),
                    jnp.full((LANES,), j, jnp.int32), drows, ioff])
                ucols_v[pl.ds((cbase + j) * EMBED_DIM + h * LANES, LANES)] = uval
                icols_v[pl.ds((cbase + j) * EMBED_DIM + h * LANES, LANES)] = ival

    def dot16(w):
        gbase = (w - (LANES // WAVE - 1)) * WAVE
        acc = jnp.zeros((LANES,), jnp.float32)
        flat0 = d16 * EMBED_DIM
        for d in range(EMBED_DIM):
            uc = plsc.load_gather(ucols_v, [flat0 + d])
            ic = plsc.load_gather(icols_v, [flat0 + d])
            acc = acc + uc * ic
        out_v[pl.ds(gbase, LANES)] = acc

    for w in range(NBUF - 1):
        fire(w, w)
    n_steps = (n_waves - (n_waves % NBUF)) // NBUF

    def pipe(step, carry):
        w0 = step * NBUF
        for p in range(NBUF):
            w = w0 + p

            @pl.when(w + NBUF - 1 < n_waves)
            def _():
                fire(w + NBUF - 1, (p + NBUF - 1) % NBUF)

            drain_extract(w, p)

            @pl.when(w % (LANES // WAVE) == (LANES // WAVE - 1))
            def _():
                dot16(w)
        return carry

    lax.fori_loop(0, n_steps, pipe, 0)
    for w in range(n_steps * NBUF, n_waves):
        drain_extract(w, w % NBUF)
        if w % (LANES // WAVE) == (LANES // WAVE - 1):
            dot16(w)
    pltpu.sync_copy(out_v, out_hbm.at[pl.ds(base, bpw)])


@jax.jit
def kernel(users, items, user_table, item_table):
    users = users.astype(jnp.int32)
    items = items.astype(jnp.int32)
    ut = user_table.T
    it = item_table.T
    batch = users.shape[0]
    bpw = batch // NUM_WORKERS
    mesh = plsc.VectorSubcoreMesh(core_axis_name="c", subcore_axis_name="s")
    run = pl.kernel(
        _mf_body,
        out_type=jax.ShapeDtypeStruct((batch,), jnp.float32),
        mesh=mesh,
        scratch_types=[
            pltpu.VMEM((bpw + LANES,), jnp.int32),
            pltpu.VMEM((bpw + LANES,), jnp.int32),
            pltpu.VMEM((NBUF, WAVE, EMBED_DIM, PANEL), jnp.float32),
            pltpu.VMEM((NBUF, WAVE, EMBED_DIM, PANEL), jnp.float32),
            pltpu.VMEM((LANES * EMBED_DIM,), jnp.float32),
            pltpu.VMEM((LANES * EMBED_DIM,), jnp.float32),
            pltpu.VMEM((bpw,), jnp.float32),
            *([pltpu.SemaphoreType.DMA] * (2 * NBUF)),
        ],
        compiler_params=pltpu.CompilerParams(
            needs_layout_passes=False, use_tc_tiling_on_sc=True),
    )
    return run(users, items, ut, it)


# speedup vs baseline: 4.3610x; 1.0515x over previous
import functools

import jax
import jax.numpy as jnp
from jax import lax
from jax.experimental import pallas as pl
from jax.experimental.pallas import tpu as pltpu
from jax.experimental.pallas import tpu_sc as plsc

LANES = 16
EMBED_DIM = 32
NUM_CORES = 2
NUM_SUBCORES = 16
NUM_WORKERS = NUM_CORES * NUM_SUBCORES
WAVE = 2
NBUF = 7
PANEL = 128


def _mf_body(users_hbm, items_hbm, ut_hbm, it_hbm, out_hbm,
             uidx_v, iidx_v, upan_v, ipan_v, ucols_v, icols_v, out_v,
             *sems):
    sem_u = sems[:NBUF]
    sem_i = sems[NBUF:]
    batch = users_hbm.shape[0]
    bpw = batch // NUM_WORKERS
    wid = lax.axis_index("s") * NUM_CORES + lax.axis_index("c")
    base = wid * bpw
    n_waves = bpw // WAVE

    pltpu.sync_copy(users_hbm.at[pl.ds(base, bpw)], uidx_v.at[pl.ds(0, bpw)])
    pltpu.sync_copy(items_hbm.at[pl.ds(base, bpw)], iidx_v.at[pl.ds(0, bpw)])

    d16 = lax.iota(jnp.int32, LANES)

    def fire(w, slot):
        uvec = uidx_v[pl.ds(w * WAVE, LANES)]
        ivec = iidx_v[pl.ds(w * WAVE, LANES)]
        for j in range(WAVE):
            uc0 = pl.multiple_of((uvec[j] >> 7) << 7, PANEL)
            ic0 = pl.multiple_of((ivec[j] >> 7) << 7, PANEL)
            pltpu.async_copy(
                ut_hbm.at[:, pl.ds(uc0, PANEL)], upan_v.at[slot, j], sem_u[slot])
            pltpu.async_copy(
                it_hbm.at[:, pl.ds(ic0, PANEL)], ipan_v.at[slot, j], sem_i[slot])

    def drain_extract(w, slot):
        uvec = uidx_v[pl.ds(w * WAVE, LANES)]
        ivec = iidx_v[pl.ds(w * WAVE, LANES)]
        for j in range(WAVE):
            pltpu.make_async_copy(
                ut_hbm.at[:, pl.ds(0, PANEL)], upan_v.at[slot, j],
                sem_u[slot]).wait()
            pltpu.make_async_copy(
                it_hbm.at[:, pl.ds(0, PANEL)], ipan_v.at[slot, j],
                sem_i[slot]).wait()
        cbase = (w % (LANES // WAVE)) * WAVE
        for j in range(WAVE):
            uoff = jnp.full((LANES,), uvec[j] & 127, jnp.int32)
            ioff = jnp.full((LANES,), ivec[j] & 127, jnp.int32)
            for h in range(EMBED_DIM // LANES):
                drows = h * LANES + d16
                uval = plsc.load_gather(upan_v, [
                    jnp.full((LANES,), slot, jnp.int32),
                    jnp.full((LANES,), j, jnp.int32), drows, uoff])
                ival = plsc.load_gather(ipan_v, [
                    jnp.full((LANES,), slot, jnp.int32),
                    jnp.full((LANES,), j, jnp.int32), drows, ioff])
                ucols_v[pl.ds((cbase + j) * EMBED_DIM + h * LANES, LANES)] = uval
                icols_v[pl.ds((cbase + j) * EMBED_DIM + h * LANES, LANES)] = ival

    def dot16(w):
        gbase = (w - (LANES // WAVE - 1)) * WAVE
        acc = jnp.zeros((LANES,), jnp.float32)
        flat0 = d16 * EMBED_DIM
        for d in range(EMBED_DIM):
            uc = plsc.load_gather(ucols_v, [flat0 + d])
            ic = plsc.load_gather(icols_v, [flat0 + d])
            acc = acc + uc * ic
        out_v[pl.ds(gbase, LANES)] = acc

    for w in range(NBUF - 1):
        fire(w, w)
    n_steps = (n_waves - (n_waves % NBUF)) // NBUF

    def pipe(step, carry):
        w0 = step * NBUF
        for p in range(NBUF):
            w = w0 + p

            @pl.when(w + NBUF - 1 < n_waves)
            def _():
                fire(w + NBUF - 1, (p + NBUF - 1) % NBUF)

            drain_extract(w, p)

            @pl.when(w % (LANES // WAVE) == (LANES // WAVE - 1))
            def _():
                dot16(w)
        return carry

    lax.fori_loop(0, n_steps, pipe, 0)
    for w in range(n_steps * NBUF, n_waves):
        drain_extract(w, w % NBUF)
        if w % (LANES // WAVE) == (LANES // WAVE - 1):
            dot16(w)
    pltpu.sync_copy(out_v, out_hbm.at[pl.ds(base, bpw)])


@jax.jit
def kernel(users, items, user_table, item_table):
    users = users.astype(jnp.int32)
    items = items.astype(jnp.int32)
    ut = user_table.T
    it = item_table.T
    batch = users.shape[0]
    bpw = batch // NUM_WORKERS
    mesh = plsc.VectorSubcoreMesh(core_axis_name="c", subcore_axis_name="s")
    run = pl.kernel(
        _mf_body,
        out_type=jax.ShapeDtypeStruct((batch,), jnp.float32),
        mesh=mesh,
        scratch_types=[
            pltpu.VMEM((bpw + LANES,), jnp.int32),
            pltpu.VMEM((bpw + LANES,), jnp.int32),
            pltpu.VMEM((NBUF, WAVE, EMBED_DIM, PANEL), jnp.float32),
            pltpu.VMEM((NBUF, WAVE, EMBED_DIM, PANEL), jnp.float32),
            pltpu.VMEM((LANES * EMBED_DIM,), jnp.float32),
            pltpu.VMEM((LANES * EMBED_DIM,), jnp.float32),
            pltpu.VMEM((bpw,), jnp.float32),
            *([pltpu.SemaphoreType.DMA] * (2 * NBUF)),
        ],
        compiler_params=pltpu.CompilerParams(
            needs_layout_passes=False, use_tc_tiling_on_sc=True),
    )
    return run(users, items, ut, it)
